# baseline (speedup 1.0000x reference)
import functools

import numpy as np
import jax
import jax.numpy as jnp
from jax import lax
from jax.experimental import pallas as pl
from jax.experimental.pallas import tpu as pltpu
from jax.experimental.pallas import tpu_sc as plsc

_ALPHA = 4
_C, _T, _H, _W = 3, 32, 384, 384
_N_SLOW = _T // _ALPHA
_FRAME = _H * _W
_QUART = _FRAME // 4
_NW = 32
_N_CHUNKS = _C * _N_SLOW * 4
_PER_W = _N_CHUNKS // _NW


def _sc_slow_body(frames_hbm, out_hbm, buf0, buf1, sem_r0, sem_r1, sem_w0, sem_w1):
    wid = lax.axis_index("s") * 2 + lax.axis_index("c")
    bufs = (buf0, buf1)
    rsems = (sem_r0, sem_r1)
    wsems = (sem_w0, sem_w1)

    def offsets(k):
        j = wid + _NW * k
        chunk = j // 4
        q = j - chunk * 4
        c = chunk // _N_SLOW
        s = chunk - c * _N_SLOW
        idx_s = (s * (_T - 1)) // (_N_SLOW - 1)
        src = (c * _T + idx_s) * _FRAME + q * _QUART
        dst = j * _QUART
        return src, dst

    reads, writes = [None, None], [None, None]
    dsts = [None, None]
    for k in range(_PER_W):
        b = k % 2
        src, dst = offsets(k)
        if writes[b] is not None:
            writes[b].wait()
        rc = pltpu.make_async_copy(frames_hbm.at[pl.ds(src, _QUART)], bufs[b], rsems[b])
        rc.start()
        reads[b], dsts[b] = rc, dst
        ob = 1 - b
        if reads[ob] is not None:
            reads[ob].wait()
            wc = pltpu.make_async_copy(bufs[ob], out_hbm.at[pl.ds(dsts[ob], _QUART)], wsems[ob])
            wc.start()
            writes[ob] = wc
            reads[ob] = None
    for b in range(2):
        if reads[b] is not None:
            reads[b].wait()
            wc = pltpu.make_async_copy(bufs[b], out_hbm.at[pl.ds(dsts[b], _QUART)], wsems[b])
            wc.start()
            writes[b] = wc
    for b in range(2):
        if writes[b] is not None:
            writes[b].wait()


_sc_slow = functools.partial(
    pl.kernel,
    mesh=plsc.VectorSubcoreMesh(core_axis_name="c", subcore_axis_name="s"),
    out_type=jax.ShapeDtypeStruct((_C * _N_SLOW * _FRAME,), jnp.float32),
    scratch_types=[
        pltpu.VMEM((_QUART,), jnp.float32),
        pltpu.VMEM((_QUART,), jnp.float32),
        pltpu.SemaphoreType.DMA,
        pltpu.SemaphoreType.DMA,
        pltpu.SemaphoreType.DMA,
        pltpu.SemaphoreType.DMA,
    ],
)(_sc_slow_body)


def _copy_body(frames_ref, fast_ref):
    fast_ref[...] = frames_ref[...]


def kernel(frames):
    C, T, H, W = frames.shape
    assert (C, T, H, W) == (_C, _T, _H, _W)
    n_slow = T // _ALPHA
    idx = np.linspace(0.0, T - 1, n_slow).astype(np.int32)
    assert all(int(i) == (s * (T - 1)) // (n_slow - 1) for s, i in enumerate(idx))

    slow_flat = _sc_slow(frames.reshape(-1))
    slow = slow_flat.reshape(C, n_slow, H, W)

    fast = pl.pallas_call(
        _copy_body,
        grid=(n_slow,),
        in_specs=[pl.BlockSpec((C, _ALPHA, H, W), lambda s: (0, s, 0, 0))],
        out_specs=pl.BlockSpec((C, _ALPHA, H, W), lambda s: (0, s, 0, 0)),
        out_shape=jax.ShapeDtypeStruct((C, T, H, W), frames.dtype),
    )(frames)
    return (slow, fast)


# speedup vs baseline: 2.1747x; 2.1747x over previous
import functools

import numpy as np
import jax
import jax.numpy as jnp
from jax import lax
from jax.experimental import pallas as pl
from jax.experimental.pallas import tpu as pltpu
from jax.experimental.pallas import tpu_sc as plsc

_ALPHA = 4
_C, _T, _H, _W = 3, 32, 384, 384
_N_SLOW = _T // _ALPHA
_QROWS = _H // 4
_NW = 32
_N_CHUNKS = _C * _N_SLOW * 4
_PER_W = _N_CHUNKS // _NW


def _sc_slow_body(frames_hbm, out_hbm, buf0, buf1, sem_r0, sem_r1, sem_w0, sem_w1):
    wid = lax.axis_index("s") * 2 + lax.axis_index("c")
    bufs = (buf0, buf1)
    rsems = (sem_r0, sem_r1)
    wsems = (sem_w0, sem_w1)

    def slices(k):
        j = wid + _NW * k
        chunk = j // 4
        q = j - chunk * 4
        c = chunk // _N_SLOW
        s = chunk - c * _N_SLOW
        idx_s = (s * (_T - 1)) // (_N_SLOW - 1)
        src = frames_hbm.at[c, idx_s, pl.ds(q * _QROWS, _QROWS), :]
        dst = out_hbm.at[c, s, pl.ds(q * _QROWS, _QROWS), :]
        return src, dst

    reads, writes, dsts = [None, None], [None, None], [None, None]
    for k in range(_PER_W):
        b = k % 2
        src, dst = slices(k)
        if writes[b] is not None:
            writes[b].wait()
        rc = pltpu.make_async_copy(src, bufs[b], rsems[b])
        rc.start()
        reads[b], dsts[b] = rc, dst
        ob = 1 - b
        if reads[ob] is not None:
            reads[ob].wait()
            wc = pltpu.make_async_copy(bufs[ob], dsts[ob], wsems[ob])
            wc.start()
            writes[ob] = wc
            reads[ob] = None
    for b in range(2):
        if reads[b] is not None:
            reads[b].wait()
            wc = pltpu.make_async_copy(bufs[b], dsts[b], wsems[b])
            wc.start()
            writes[b] = wc
    for b in range(2):
        if writes[b] is not None:
            writes[b].wait()


_sc_slow = functools.partial(
    pl.kernel,
    mesh=plsc.VectorSubcoreMesh(core_axis_name="c", subcore_axis_name="s"),
    out_type=jax.ShapeDtypeStruct((_C, _N_SLOW, _H, _W), jnp.float32),
    scratch_types=[
        pltpu.VMEM((_QROWS, _W), jnp.float32),
        pltpu.VMEM((_QROWS, _W), jnp.float32),
        pltpu.SemaphoreType.DMA,
        pltpu.SemaphoreType.DMA,
        pltpu.SemaphoreType.DMA,
        pltpu.SemaphoreType.DMA,
    ],
)(_sc_slow_body)


def _copy_body(frames_ref, fast_ref):
    fast_ref[...] = frames_ref[...]


def kernel(frames):
    C, T, H, W = frames.shape
    assert (C, T, H, W) == (_C, _T, _H, _W)
    n_slow = T // _ALPHA
    idx = np.linspace(0.0, T - 1, n_slow).astype(np.int32)
    assert all(int(i) == (s * (T - 1)) // (n_slow - 1) for s, i in enumerate(idx))

    slow = _sc_slow(frames)

    fast = pl.pallas_call(
        _copy_body,
        grid=(n_slow,),
        in_specs=[pl.BlockSpec((C, _ALPHA, H, W), lambda s: (0, s, 0, 0))],
        out_specs=pl.BlockSpec((C, _ALPHA, H, W), lambda s: (0, s, 0, 0)),
        out_shape=jax.ShapeDtypeStruct((C, T, H, W), frames.dtype),
    )(frames)
    return (slow, fast)


# speedup vs baseline: 3.4340x; 1.5791x over previous
import numpy as np
import jax
import jax.numpy as jnp
from jax.experimental import pallas as pl
from jax.experimental.pallas import tpu as pltpu

_ALPHA = 4
_GROUP = 8
_SLOW_PER_GROUP = _GROUP // _ALPHA


def _pack_body(frames_ref, slow_ref, fast_ref):
    g = pl.program_id(0)
    n_slow = pl.num_programs(0) * _SLOW_PER_GROUP
    T = n_slow * _ALPHA
    fast_ref[...] = frames_ref[...]
    for u in range(_SLOW_PER_GROUP):
        s = g * _SLOW_PER_GROUP + u
        off = (s * (T - 1)) // (n_slow - 1) - _GROUP * g
        slow_ref[:, pl.ds(u, 1), :, :] = frames_ref[:, pl.ds(off, 1), :, :]


def kernel(frames):
    C, T, H, W = frames.shape
    n_slow = T // _ALPHA
    n_groups = T // _GROUP
    idx = np.linspace(0.0, T - 1, n_slow).astype(np.int32)
    assert all(int(i) == (s * (T - 1)) // (n_slow - 1) for s, i in enumerate(idx))
    assert all(_GROUP * (s // _SLOW_PER_GROUP) <= int(i) < _GROUP * (s // _SLOW_PER_GROUP + 1)
               for s, i in enumerate(idx))

    def group_map(g):
        return (0, g, 0, 0)

    slow, fast = pl.pallas_call(
        _pack_body,
        grid=(n_groups,),
        in_specs=[pl.BlockSpec((C, _GROUP, H, W), group_map)],
        out_specs=[
            pl.BlockSpec((C, _SLOW_PER_GROUP, H, W), group_map),
            pl.BlockSpec((C, _GROUP, H, W), group_map),
        ],
        out_shape=[
            jax.ShapeDtypeStruct((C, n_slow, H, W), frames.dtype),
            jax.ShapeDtypeStruct((C, T, H, W), frames.dtype),
        ],
        compiler_params=pltpu.CompilerParams(vmem_limit_bytes=100 * 1024 * 1024),
    )(frames)
    return (slow, fast)
